# Initial kernel scaffold; baseline (speedup 1.0000x reference)
import jax
import jax.numpy as jnp
from jax.experimental import pallas as pl


def kernel(x, W, temp):
    raise NotImplementedError("write your pallas kernel here")


# speedup vs baseline: 1.7333x; 1.7333x over previous
import jax
import jax.numpy as jnp
from jax import lax
from jax.experimental import pallas as pl

D_MODEL = 768
E = 64


def _router_body(x_ref, w_ref, p_ref, i_ref, mask_ref, psum_ref, msum_ref, ent_ref):
    T = x_ref.shape[0]
    logits = lax.dot_general(
        x_ref[:], w_ref[:], (((1,), (1,)), ((), ())),
        preferred_element_type=jnp.float32)
    m = jnp.max(logits, axis=-1, keepdims=True)
    e = jnp.exp(logits - m)
    s = jnp.sum(e, axis=-1, keepdims=True)
    p = e / s

    lane = lax.broadcasted_iota(jnp.int32, (T, E), 1)
    m1 = jnp.max(p, axis=-1, keepdims=True)
    i1 = jnp.min(jnp.where(p == m1, lane, E), axis=-1, keepdims=True)
    hit1 = lane == i1
    pm = jnp.where(hit1, -1.0, p)
    m2 = jnp.max(pm, axis=-1, keepdims=True)
    i2 = jnp.min(jnp.where(pm == m2, lane, E), axis=-1, keepdims=True)
    hit2 = lane == i2
    mask = (hit1 | hit2).astype(jnp.float32)

    p_ref[:] = jnp.concatenate([m1, m2], axis=-1)
    i_ref[:] = jnp.concatenate([i1, i2], axis=-1)
    mask_ref[:] = mask

    plogp = p * jnp.log(jnp.maximum(p, 1e-8))

    @pl.when(pl.program_id(0) == 0)
    def _init():
        psum_ref[:] = jnp.zeros_like(psum_ref)
        msum_ref[:] = jnp.zeros_like(msum_ref)
        ent_ref[:] = jnp.zeros_like(ent_ref)

    psum_ref[:] += jnp.sum(p, axis=0, keepdims=True)
    msum_ref[:] += jnp.sum(mask, axis=0, keepdims=True)
    ent_ref[:] += jnp.sum(plogp).reshape(1, 1)


def kernel(x, W, temp):
    B, S, D = x.shape
    N = B * S
    t = jnp.clip(temp, 0.1, 5.0)
    w = W / t
    xf = x.reshape(N, D)
    T = 2048
    grid = N // T

    outs = pl.pallas_call(
        _router_body,
        grid=(grid,),
        in_specs=[
            pl.BlockSpec((T, D), lambda i: (i, 0)),
            pl.BlockSpec((E, D), lambda i: (0, 0)),
        ],
        out_specs=[
            pl.BlockSpec((T, 2), lambda i: (i, 0)),
            pl.BlockSpec((T, 2), lambda i: (i, 0)),
            pl.BlockSpec((T, E), lambda i: (i, 0)),
            pl.BlockSpec((1, E), lambda i: (0, 0)),
            pl.BlockSpec((1, E), lambda i: (0, 0)),
            pl.BlockSpec((1, 1), lambda i: (0, 0)),
        ],
        out_shape=[
            jax.ShapeDtypeStruct((N, 2), jnp.float32),
            jax.ShapeDtypeStruct((N, 2), jnp.int32),
            jax.ShapeDtypeStruct((N, E), jnp.float32),
            jax.ShapeDtypeStruct((1, E), jnp.float32),
            jax.ShapeDtypeStruct((1, E), jnp.float32),
            jax.ShapeDtypeStruct((1, 1), jnp.float32),
        ],
    )(xf, w)

    p2, idx2, mask, psum, msum, ent = outs
    expert_probs = p2.reshape(B, S, 2)
    expert_indices = idx2.reshape(B, S, 2)
    expert_mask = mask.reshape(B, S, E)
    denom = jnp.float32(N)
    importance = psum[0] / denom
    load = msum[0] / (denom + 1e-6)
    aux_load_loss = jnp.sum(importance * load) * E * 0.01
    router_entropy = (-ent[0, 0] / denom) * 0.01
    return expert_probs, expert_indices, expert_mask, aux_load_loss, router_entropy


# speedup vs baseline: 1.8097x; 1.0441x over previous
import jax
import jax.numpy as jnp
from jax import lax
from jax.experimental import pallas as pl

D_MODEL = 768
E = 64


def _router_body(x_ref, w_ref, p_ref, i_ref, mask_ref, psum_ref, msum_ref, ent_ref):
    T = x_ref.shape[0]
    logits = lax.dot_general(
        x_ref[:], w_ref[:], (((1,), (1,)), ((), ())),
        preferred_element_type=jnp.float32)
    m = jnp.max(logits, axis=-1, keepdims=True)
    e = jnp.exp(logits - m)
    s = jnp.sum(e, axis=-1, keepdims=True)
    p = e / s

    lane = lax.broadcasted_iota(jnp.int32, (T, E), 1)
    m1 = jnp.max(p, axis=-1, keepdims=True)
    i1 = jnp.min(jnp.where(p == m1, lane, E), axis=-1, keepdims=True)
    hit1 = lane == i1
    pm = jnp.where(hit1, -1.0, p)
    m2 = jnp.max(pm, axis=-1, keepdims=True)
    i2 = jnp.min(jnp.where(pm == m2, lane, E), axis=-1, keepdims=True)
    hit2 = lane == i2
    mask = (hit1 | hit2).astype(jnp.float32)

    p_ref[:] = jnp.concatenate([m1, m2], axis=-1)
    i_ref[:] = jnp.concatenate([i1, i2], axis=-1)
    mask_ref[:] = mask

    plogp = p * jnp.log(jnp.maximum(p, 1e-8))

    @pl.when(pl.program_id(0) == 0)
    def _init():
        psum_ref[:] = jnp.zeros_like(psum_ref)
        msum_ref[:] = jnp.zeros_like(msum_ref)
        ent_ref[:] = jnp.zeros_like(ent_ref)

    psum_ref[:] += jnp.sum(p, axis=0, keepdims=True)
    msum_ref[:] += jnp.sum(mask, axis=0, keepdims=True)
    ent_ref[:] += jnp.sum(plogp).reshape(1, 1)


def kernel(x, W, temp):
    B, S, D = x.shape
    N = B * S
    t = jnp.clip(temp, 0.1, 5.0)
    w = W / t
    xf = x.reshape(N, D)
    T = 4096
    grid = N // T

    outs = pl.pallas_call(
        _router_body,
        grid=(grid,),
        in_specs=[
            pl.BlockSpec((T, D), lambda i: (i, 0)),
            pl.BlockSpec((E, D), lambda i: (0, 0)),
        ],
        out_specs=[
            pl.BlockSpec((T, 2), lambda i: (i, 0)),
            pl.BlockSpec((T, 2), lambda i: (i, 0)),
            pl.BlockSpec((T, E), lambda i: (i, 0)),
            pl.BlockSpec((1, E), lambda i: (0, 0)),
            pl.BlockSpec((1, E), lambda i: (0, 0)),
            pl.BlockSpec((1, 1), lambda i: (0, 0)),
        ],
        out_shape=[
            jax.ShapeDtypeStruct((N, 2), jnp.float32),
            jax.ShapeDtypeStruct((N, 2), jnp.int32),
            jax.ShapeDtypeStruct((N, E), jnp.float32),
            jax.ShapeDtypeStruct((1, E), jnp.float32),
            jax.ShapeDtypeStruct((1, E), jnp.float32),
            jax.ShapeDtypeStruct((1, 1), jnp.float32),
        ],
    )(xf, w)

    p2, idx2, mask, psum, msum, ent = outs
    expert_probs = p2.reshape(B, S, 2)
    expert_indices = idx2.reshape(B, S, 2)
    expert_mask = mask.reshape(B, S, E)
    denom = jnp.float32(N)
    importance = psum[0] / denom
    load = msum[0] / (denom + 1e-6)
    aux_load_loss = jnp.sum(importance * load) * E * 0.01
    router_entropy = (-ent[0, 0] / denom) * 0.01
    return expert_probs, expert_indices, expert_mask, aux_load_loss, router_entropy
